# Initial kernel scaffold; baseline (speedup 1.0000x reference)
import jax
import jax.numpy as jnp
from jax.experimental import pallas as pl


def kernel(input0, input1):
    raise NotImplementedError("write your pallas kernel here")


# speedup vs baseline: 1.0979x; 1.0979x over previous
import functools

import jax
import jax.numpy as jnp
from jax.experimental import pallas as pl
from jax.experimental.pallas import tpu as pltpu


def _chamfer_body(x0_ref, x1t_ref, sq0_ref, sq1_ref, out_ref, d0_ref, d1_ref,
                  *, g0, g1, n0, n1):
    i = pl.program_id(0)
    j = pl.program_id(1)

    dotv = jnp.dot(x0_ref[...], x1t_ref[...],
                   preferred_element_type=jnp.float32)
    d2 = sq0_ref[...] + sq1_ref[...] + dotv
    rmin = jnp.min(d2, axis=1)
    cmin = jnp.min(d2, axis=0)

    @pl.when(j == 0)
    def _():
        d0_ref[i, :] = rmin

    @pl.when(j > 0)
    def _():
        d0_ref[i, :] = jnp.minimum(d0_ref[i, :], rmin)

    @pl.when(i == 0)
    def _():
        d1_ref[j, :] = cmin

    @pl.when(i > 0)
    def _():
        d1_ref[j, :] = jnp.minimum(d1_ref[j, :], cmin)

    @pl.when(jnp.logical_and(i == g0 - 1, j == g1 - 1))
    def _():
        s0 = jnp.sum(jnp.maximum(d0_ref[...], 0.0)) / n0
        s1 = jnp.sum(jnp.maximum(d1_ref[...], 0.0)) / n1
        out_ref[...] = (s0 + s1).reshape(1, 1)


@jax.jit
def kernel(input0, input1):
    n0 = input0.shape[0]
    n1 = input1.shape[0]
    b0 = 1024
    b1 = 2048
    g0 = n0 // b0
    g1 = n1 // b1

    f32 = jnp.float32
    sq0 = jnp.sum(input0 * input0, axis=1, keepdims=True)
    sq1 = jnp.sum(input1 * input1, axis=1).reshape(1, n1)
    x0 = jnp.concatenate([input0, jnp.zeros((n0, 5), f32)], axis=1)
    x1t = jnp.concatenate([-2.0 * input1, jnp.zeros((n1, 5), f32)], axis=1).T

    body = functools.partial(_chamfer_body, g0=g0, g1=g1, n0=n0, n1=n1)
    out = pl.pallas_call(
        body,
        grid=(g0, g1),
        in_specs=[
            pl.BlockSpec((b0, 8), lambda i, j: (i, 0)),
            pl.BlockSpec((8, b1), lambda i, j: (0, j)),
            pl.BlockSpec((b0, 1), lambda i, j: (i, 0)),
            pl.BlockSpec((1, b1), lambda i, j: (0, j)),
        ],
        out_specs=pl.BlockSpec((1, 1), lambda i, j: (0, 0)),
        out_shape=jax.ShapeDtypeStruct((1, 1), f32),
        scratch_shapes=[
            pltpu.VMEM((g0, b0), f32),
            pltpu.VMEM((g1, b1), f32),
        ],
    )(x0, x1t, sq0, sq1)
    return out[0, 0]


# speedup vs baseline: 1.2832x; 1.1688x over previous
import functools

import jax
import jax.numpy as jnp
from jax.experimental import pallas as pl
from jax.experimental.pallas import tpu as pltpu


def _chamfer_body(x0_ref, x1t_ref, sq0_ref, sq1_ref, out_ref,
                  rowacc_ref, colacc_ref, *, b0, b1, g0, g1, n0, n1):
    i = pl.program_id(0)
    j = pl.program_id(1)

    dotv = jnp.dot(x0_ref[...], x1t_ref[...],
                   preferred_element_type=jnp.float32)

    sq0b = sq0_ref[pl.ds(i * b0, b0), :]
    sq1b = sq1_ref[:, pl.ds(j * b1, b1)]

    r = dotv + sq1b
    w = b1
    while w > 128:
        w //= 2
        r = jnp.minimum(r[:, :w], r[:, w:])
    c = dotv + sq0b
    h = b0
    while h > 8:
        h //= 2
        c = jnp.minimum(c[:h, :], c[h:, :])

    row_slice = pl.ds(i * b0, b0)
    col_slice = pl.ds(j * 8, 8)

    @pl.when(j == 0)
    def _():
        rowacc_ref[row_slice, :] = r

    @pl.when(j > 0)
    def _():
        rowacc_ref[row_slice, :] = jnp.minimum(rowacc_ref[row_slice, :], r)

    @pl.when(i == 0)
    def _():
        colacc_ref[col_slice, :] = c

    @pl.when(i > 0)
    def _():
        colacc_ref[col_slice, :] = jnp.minimum(colacc_ref[col_slice, :], c)

    @pl.when(jnp.logical_and(i == g0 - 1, j == g1 - 1))
    def _():
        rm = jnp.min(rowacc_ref[...], axis=1) + sq0_ref[:, 0]
        s0 = jnp.sum(jnp.maximum(rm, 0.0)) / n0
        s1 = 0.0
        for jj in range(g1):
            cj = colacc_ref[8 * jj:8 * jj + 8, :]
            cj = jnp.minimum(cj[:4, :], cj[4:, :])
            cj = jnp.minimum(cj[:2, :], cj[2:, :])
            cj = jnp.minimum(cj[:1, :], cj[1:, :])
            cj = cj + sq1_ref[:, b1 * jj:b1 * (jj + 1)]
            s1 = s1 + jnp.sum(jnp.maximum(cj, 0.0))
        out_ref[...] = (s0 + s1 / n1).reshape(1, 1)


@jax.jit
def kernel(input0, input1):
    n0 = input0.shape[0]
    n1 = input1.shape[0]
    b0 = 2048
    b1 = 2048
    g0 = n0 // b0
    g1 = n1 // b1

    f32 = jnp.float32
    sq0 = jnp.sum(input0 * input0, axis=1, keepdims=True)
    sq1 = jnp.sum(input1 * input1, axis=1).reshape(1, n1)
    x0 = jnp.concatenate([input0, jnp.zeros((n0, 5), f32)], axis=1)
    x1t = jnp.concatenate([-2.0 * input1, jnp.zeros((n1, 5), f32)], axis=1).T

    body = functools.partial(
        _chamfer_body, b0=b0, b1=b1, g0=g0, g1=g1, n0=n0, n1=n1)
    out = pl.pallas_call(
        body,
        grid=(g0, g1),
        in_specs=[
            pl.BlockSpec((b0, 8), lambda i, j: (i, 0)),
            pl.BlockSpec((8, b1), lambda i, j: (0, j)),
            pl.BlockSpec((n0, 1), lambda i, j: (0, 0)),
            pl.BlockSpec((1, n1), lambda i, j: (0, 0)),
        ],
        out_specs=pl.BlockSpec((1, 1), lambda i, j: (0, 0)),
        out_shape=jax.ShapeDtypeStruct((1, 1), f32),
        scratch_shapes=[
            pltpu.VMEM((n0, 128), f32),
            pltpu.VMEM((g1 * 8, b1), f32),
        ],
    )(x0, x1t, sq0, sq1)
    return out[0, 0]


# speedup vs baseline: 1.4898x; 1.1610x over previous
import functools

import jax
import jax.numpy as jnp
from jax.experimental import pallas as pl
from jax.experimental.pallas import tpu as pltpu

_CHUNK = 512


def _chamfer_body(x0_ref, x1t_ref, sq0_ref, sq1_ref, out_ref,
                  rowacc_ref, colacc_ref, *, b0, b1, g0, g1, n0, n1):
    i = pl.program_id(0)
    j = pl.program_id(1)

    x0b = x0_ref[...]
    sq0b = sq0_ref[pl.ds(i * b0, b0), :]

    r_part = None
    c_parts = []
    for k in range(b1 // _CHUNK):
        x1c = x1t_ref[:, k * _CHUNK:(k + 1) * _CHUNK]
        dk = jnp.dot(x0b, x1c, preferred_element_type=jnp.float32)
        sq1c = sq1_ref[:, pl.ds(j * b1 + k * _CHUNK, _CHUNK)]
        d2 = (dk + sq0b) + sq1c

        t = d2
        w = _CHUNK
        while w > 128:
            w //= 2
            t = jnp.minimum(t[:, :w], t[:, w:])
        r_part = t if r_part is None else jnp.minimum(r_part, t)

        c = d2
        h = b0
        while h > 8:
            h //= 2
            c = jnp.minimum(c[:h, :], c[h:, :])
        c_parts.append(c)

    c_part = jnp.concatenate(c_parts, axis=1)

    row_slice = pl.ds(i * b0, b0)
    col_slice = pl.ds(j * 8, 8)

    @pl.when(j == 0)
    def _():
        rowacc_ref[row_slice, :] = r_part

    @pl.when(j > 0)
    def _():
        rowacc_ref[row_slice, :] = jnp.minimum(
            rowacc_ref[row_slice, :], r_part)

    @pl.when(i == 0)
    def _():
        colacc_ref[col_slice, :] = c_part

    @pl.when(i > 0)
    def _():
        colacc_ref[col_slice, :] = jnp.minimum(
            colacc_ref[col_slice, :], c_part)

    @pl.when(jnp.logical_and(i == g0 - 1, j == g1 - 1))
    def _():
        rm = jnp.min(rowacc_ref[...], axis=1)
        s0 = jnp.sum(jnp.maximum(rm, 0.0)) / n0
        s1 = 0.0
        for jj in range(g1):
            cj = colacc_ref[8 * jj:8 * jj + 8, :]
            cj = jnp.minimum(cj[:4, :], cj[4:, :])
            cj = jnp.minimum(cj[:2, :], cj[2:, :])
            cj = jnp.minimum(cj[:1, :], cj[1:, :])
            s1 = s1 + jnp.sum(jnp.maximum(cj, 0.0))
        out_ref[...] = (s0 + s1 / n1).reshape(1, 1)


@jax.jit
def kernel(input0, input1):
    n0 = input0.shape[0]
    n1 = input1.shape[0]
    b0 = 2048
    b1 = 2048
    g0 = n0 // b0
    g1 = n1 // b1

    f32 = jnp.float32
    sq0 = jnp.sum(input0 * input0, axis=1, keepdims=True)
    sq1 = jnp.sum(input1 * input1, axis=1).reshape(1, n1)
    x0 = jnp.concatenate([input0, jnp.zeros((n0, 5), f32)], axis=1)
    x1t = jnp.concatenate([-2.0 * input1, jnp.zeros((n1, 5), f32)], axis=1).T

    body = functools.partial(
        _chamfer_body, b0=b0, b1=b1, g0=g0, g1=g1, n0=n0, n1=n1)
    out = pl.pallas_call(
        body,
        grid=(g0, g1),
        in_specs=[
            pl.BlockSpec((b0, 8), lambda i, j: (i, 0)),
            pl.BlockSpec((8, b1), lambda i, j: (0, j)),
            pl.BlockSpec((n0, 1), lambda i, j: (0, 0)),
            pl.BlockSpec((1, n1), lambda i, j: (0, 0)),
        ],
        out_specs=pl.BlockSpec((1, 1), lambda i, j: (0, 0)),
        out_shape=jax.ShapeDtypeStruct((1, 1), f32),
        scratch_shapes=[
            pltpu.VMEM((n0, 128), f32),
            pltpu.VMEM((g1 * 8, b1), f32),
        ],
    )(x0, x1t, sq0, sq1)
    return out[0, 0]
